# speedup vs baseline: 1.4577x; 1.4577x over previous
import functools

import jax
import jax.numpy as jnp
from jax.experimental import pallas as pl
from jax.experimental.pallas import tpu as pltpu

NUM_EXPERTS = 8
TOP_K = 2
HIDDEN = 1024
INTER = 768
BLK = 256


def _ffn_block_kernel(eid_ref, xs_ref, gp_ref, up_ref, dp_ref, out_ref):
    x = xs_ref[...]
    g = jax.lax.dot_general(x, gp_ref[0], (((1,), (1,)), ((), ())),
                            preferred_element_type=jnp.float32)
    u = jax.lax.dot_general(x, up_ref[0], (((1,), (1,)), ((), ())),
                            preferred_element_type=jnp.float32)
    h = (g * jax.nn.sigmoid(g) * u).astype(jnp.bfloat16)
    out_ref[...] = jax.lax.dot_general(h, dp_ref[0], (((1,), (1,)), ((), ())),
                                       preferred_element_type=jnp.float32)


def _grouped_ffn(xs, gate_proj, up_proj, down_proj, expert_ids, nblk):
    grid_spec = pltpu.PrefetchScalarGridSpec(
        num_scalar_prefetch=1,
        grid=(nblk,),
        in_specs=[
            pl.BlockSpec((BLK, HIDDEN), lambda b, eid: (b, 0)),
            pl.BlockSpec((1, INTER, HIDDEN), lambda b, eid: (eid[b], 0, 0)),
            pl.BlockSpec((1, INTER, HIDDEN), lambda b, eid: (eid[b], 0, 0)),
            pl.BlockSpec((1, HIDDEN, INTER), lambda b, eid: (eid[b], 0, 0)),
        ],
        out_specs=pl.BlockSpec((BLK, HIDDEN), lambda b, eid: (b, 0)),
    )
    return pl.pallas_call(
        _ffn_block_kernel,
        grid_spec=grid_spec,
        out_shape=jax.ShapeDtypeStruct((xs.shape[0], HIDDEN), jnp.float32),
    )(expert_ids, xs, gate_proj, up_proj, down_proj)


def kernel(hidden_states, gate_weight, gate_proj, up_proj, down_proj):
    b, s, h = hidden_states.shape
    x = hidden_states.reshape(-1, h)
    T = x.shape[0]
    nblk = T * TOP_K // BLK + NUM_EXPERTS
    padded = nblk * BLK

    logits = x @ gate_weight.T
    probs = jax.nn.softmax(logits.astype(jnp.float32), axis=1)
    w, sel = jax.lax.top_k(probs, TOP_K)
    w = w / jnp.sum(w, axis=-1, keepdims=True)

    sflat = sel.reshape(-1)
    onehot = (sflat[:, None] == jnp.arange(NUM_EXPERTS)[None, :]).astype(jnp.float32)
    incl = jnp.cumsum(onehot, axis=0)
    counts = incl[-1]
    padded_counts = jnp.ceil(counts / BLK) * BLK
    cum_padded = jnp.cumsum(padded_counts)
    pad_off = cum_padded - padded_counts
    rank = jnp.sum(onehot * incl, axis=1) - 1.0
    pos = (jnp.sum(onehot * pad_off[None, :], axis=1) + rank).astype(jnp.int32)

    bstart = jnp.arange(nblk) * BLK
    expert_ids = jnp.sum((bstart[:, None] >= cum_padded[None, :]).astype(jnp.int32), axis=1)
    expert_ids = jnp.minimum(expert_ids, NUM_EXPERTS - 1)

    xs = jnp.zeros((padded, HIDDEN), jnp.bfloat16)
    src = jnp.repeat(jnp.arange(T), TOP_K)
    xs = xs.at[pos].set(x[src].astype(jnp.bfloat16))

    ybuf = _grouped_ffn(xs, gate_proj.astype(jnp.bfloat16),
                        up_proj.astype(jnp.bfloat16),
                        down_proj.astype(jnp.bfloat16), expert_ids, nblk)

    posr = pos.reshape(T, TOP_K)
    out = w[:, 0:1] * ybuf[posr[:, 0]] + w[:, 1:2] * ybuf[posr[:, 1]]
    return out.reshape(b, s, h), logits


# speedup vs baseline: 2.4499x; 1.6807x over previous
import dataclasses
import functools

import jax
import jax.numpy as jnp
from jax import lax
from jax.experimental import pallas as pl
from jax.experimental.pallas import tpu as pltpu
from jax.experimental.pallas import tpu_sc as plsc

NUM_EXPERTS = 8
TOP_K = 2
HIDDEN = 1024
INTER = 768
BLK = 256

NC = 2
NS = 16
NW = NC * NS


def _ffn_block_kernel(eid_ref, xs_ref, gp_ref, up_ref, dp_ref, out_ref):
    x = xs_ref[...].astype(jnp.bfloat16)
    gp = gp_ref[0].astype(jnp.bfloat16)
    up = up_ref[0].astype(jnp.bfloat16)
    dp = dp_ref[0].astype(jnp.bfloat16)
    g = lax.dot_general(x, gp, (((1,), (1,)), ((), ())),
                        preferred_element_type=jnp.float32)
    u = lax.dot_general(x, up, (((1,), (1,)), ((), ())),
                        preferred_element_type=jnp.float32)
    h = (g * jax.nn.sigmoid(g) * u).astype(jnp.bfloat16)
    out_ref[...] = lax.dot_general(h, dp, (((1,), (1,)), ((), ())),
                                   preferred_element_type=jnp.float32)


def _grouped_ffn(xs, gate_proj, up_proj, down_proj, expert_ids, nblk):
    grid_spec = pltpu.PrefetchScalarGridSpec(
        num_scalar_prefetch=1,
        grid=(nblk,),
        in_specs=[
            pl.BlockSpec((BLK, HIDDEN), lambda b, eid: (b, 0)),
            pl.BlockSpec((1, INTER, HIDDEN), lambda b, eid: (eid[b], 0, 0)),
            pl.BlockSpec((1, INTER, HIDDEN), lambda b, eid: (eid[b], 0, 0)),
            pl.BlockSpec((1, HIDDEN, INTER), lambda b, eid: (eid[b], 0, 0)),
        ],
        out_specs=pl.BlockSpec((BLK, HIDDEN), lambda b, eid: (b, 0)),
    )
    return pl.pallas_call(
        _ffn_block_kernel,
        grid_spec=grid_spec,
        out_shape=jax.ShapeDtypeStruct((xs.shape[0], HIDDEN), jnp.float32),
    )(expert_ids, xs, gate_proj, up_proj, down_proj)


def _sc_mesh():
    return plsc.VectorSubcoreMesh(core_axis_name="c", subcore_axis_name="s",
                                  num_cores=NC, num_subcores=NS)


def _sc_params():
    cp = pltpu.CompilerParams()
    if "needs_layout_passes" in pltpu.CompilerParams.__dataclass_fields__:
        cp = dataclasses.replace(cp, needs_layout_passes=False)
    return cp


def _scatter_body(tpw, x_hbm, idx_hbm, xs_hbm, rows_v, idx_v, sem):
    wid = lax.axis_index("c") * NS + lax.axis_index("s")
    base = wid * tpw
    pltpu.sync_copy(x_hbm.at[pl.ds(base, tpw)], rows_v)
    pltpu.sync_copy(idx_hbm.at[wid], idx_v)
    pltpu.async_copy(rows_v, xs_hbm.at[idx_v.at[0]], sem).wait()
    pltpu.async_copy(rows_v, xs_hbm.at[idx_v.at[1]], sem).wait()


def _dispatch_scatter(x, idx3, padded):
    t = x.shape[0]
    tpw = t // NW
    return pl.kernel(
        functools.partial(_scatter_body, tpw),
        out_type=jax.ShapeDtypeStruct((padded, HIDDEN), jnp.float32),
        mesh=_sc_mesh(),
        scratch_types=[
            pltpu.VMEM((tpw, HIDDEN), jnp.float32),
            pltpu.VMEM((TOP_K, tpw), jnp.int32),
            pltpu.SemaphoreType.DMA,
        ],
    )(x, idx3)


def _combine_body(tpw, chunk, y_hbm, idx_hbm, w_hbm, out_hbm,
                  buf0, buf1, idx_v, w_v, sem):
    wid = lax.axis_index("c") * NS + lax.axis_index("s")
    base = wid * tpw
    pltpu.sync_copy(idx_hbm.at[wid], idx_v)
    pltpu.sync_copy(w_hbm.at[wid], w_v)
    nchunks = tpw // chunk
    zero16 = jnp.zeros((16,), jnp.int32)
    one16 = jnp.ones((16,), jnp.int32)
    for c in range(nchunks):
        pltpu.async_copy(y_hbm.at[idx_v.at[0, pl.ds(c * chunk, chunk)]],
                         buf0, sem).wait()
        pltpu.async_copy(y_hbm.at[idx_v.at[1, pl.ds(c * chunk, chunk)]],
                         buf1, sem).wait()

        def row_body(r, _, c=c):
            rsplat = jnp.full((16,), c * chunk + r, jnp.int32)
            w0 = plsc.load_gather(w_v, [zero16, rsplat])
            w1 = plsc.load_gather(w_v, [one16, rsplat])
            for l in range(HIDDEN // 16):
                sl = pl.ds(l * 16, 16)
                buf0[r, sl] = buf0[r, sl] * w0 + buf1[r, sl] * w1
            return 0

        lax.fori_loop(0, chunk, row_body, 0)
        pltpu.sync_copy(buf0, out_hbm.at[pl.ds(base + c * chunk, chunk)])


def _combine(ybuf, idx3, w3, t):
    tpw = t // NW
    chunk = 32
    return pl.kernel(
        functools.partial(_combine_body, tpw, chunk),
        out_type=jax.ShapeDtypeStruct((t, HIDDEN), jnp.float32),
        mesh=_sc_mesh(),
        scratch_types=[
            pltpu.VMEM((chunk, HIDDEN), jnp.float32),
            pltpu.VMEM((chunk, HIDDEN), jnp.float32),
            pltpu.VMEM((TOP_K, tpw), jnp.int32),
            pltpu.VMEM((TOP_K, tpw), jnp.float32),
            pltpu.SemaphoreType.DMA,
        ],
        compiler_params=_sc_params(),
    )(ybuf, idx3, w3)


def kernel(hidden_states, gate_weight, gate_proj, up_proj, down_proj):
    b, s, h = hidden_states.shape
    x = hidden_states.reshape(-1, h)
    T = x.shape[0]
    nblk = T * TOP_K // BLK + NUM_EXPERTS
    padded = nblk * BLK

    logits = x @ gate_weight.T
    probs = jax.nn.softmax(logits.astype(jnp.float32), axis=1)
    w, sel = jax.lax.top_k(probs, TOP_K)
    w = w / jnp.sum(w, axis=-1, keepdims=True)

    sflat = sel.reshape(-1)
    onehot = (sflat[:, None] == jnp.arange(NUM_EXPERTS)[None, :]).astype(jnp.float32)
    incl = jnp.cumsum(onehot, axis=0)
    counts = incl[-1]
    padded_counts = jnp.ceil(counts / BLK) * BLK
    cum_padded = jnp.cumsum(padded_counts)
    pad_off = cum_padded - padded_counts
    rank = jnp.sum(onehot * incl, axis=1) - 1.0
    pos = (jnp.sum(onehot * pad_off[None, :], axis=1) + rank).astype(jnp.int32)

    bstart = jnp.arange(nblk) * BLK
    expert_ids = jnp.sum((bstart[:, None] >= cum_padded[None, :]).astype(jnp.int32), axis=1)
    expert_ids = jnp.minimum(expert_ids, NUM_EXPERTS - 1)

    idx3 = pos.reshape(NW, T // NW, TOP_K).transpose(0, 2, 1)
    w3 = w.reshape(NW, T // NW, TOP_K).transpose(0, 2, 1)

    xs = _dispatch_scatter(x, idx3, padded)
    ybuf = _grouped_ffn(xs, gate_proj, up_proj, down_proj, expert_ids, nblk)
    out = _combine(ybuf, idx3, w3, T)
    return out.reshape(b, s, h), logits


# speedup vs baseline: 2.4514x; 1.0006x over previous
import dataclasses
import functools

import jax
import jax.numpy as jnp
from jax import lax
from jax.experimental import pallas as pl
from jax.experimental.pallas import tpu as pltpu
from jax.experimental.pallas import tpu_sc as plsc

NUM_EXPERTS = 8
TOP_K = 2
HIDDEN = 1024
INTER = 768
BLK = 256

NC = 2
NS = 16
NW = NC * NS


def _meta_kernel(nblk, lg_ref, pos_ref, wts_ref, eids_ref):
    t = lg_ref.shape[0]
    lg = lg_ref[...]
    m = jnp.max(lg, axis=1, keepdims=True)
    ex = jnp.exp(lg - m)
    p = ex / jnp.sum(ex, axis=1, keepdims=True)

    idx = jax.lax.broadcasted_iota(jnp.int32, (t, NUM_EXPERTS), 1)
    m1 = jnp.max(p, axis=1, keepdims=True)
    i1 = jnp.min(jnp.where(p == m1, idx, NUM_EXPERTS), axis=1, keepdims=True)
    masked = jnp.where(idx == i1, -1.0, p)
    m2 = jnp.max(masked, axis=1, keepdims=True)
    i2 = jnp.min(jnp.where(masked == m2, idx, NUM_EXPERTS), axis=1, keepdims=True)
    wsum = m1 + m2
    w0 = m1 / wsum
    w1 = m2 / wsum

    oh0 = (idx == i1).astype(jnp.float32)
    oh1 = (idx == i2).astype(jnp.float32)
    ohsum = oh0 + oh1

    ci = jax.lax.broadcasted_iota(jnp.int32, (256, 256), 0)
    cj = jax.lax.broadcasted_iota(jnp.int32, (256, 256), 1)
    lstrict = (cj < ci).astype(jnp.bfloat16)
    nchunks = t // 256
    carry = jnp.zeros((1, NUM_EXPERTS), jnp.float32)
    excls = []
    for c in range(nchunks):
        oh_c = jax.lax.slice(ohsum, (c * 256, 0), ((c + 1) * 256, NUM_EXPERTS))
        excl_c = jax.lax.dot_general(lstrict, oh_c.astype(jnp.bfloat16),
                                     (((1,), (0,)), ((), ())),
                                     preferred_element_type=jnp.float32) + carry
        excls.append(excl_c)
        carry = carry + jnp.sum(oh_c, axis=0, keepdims=True)
    counts = carry

    padded = jnp.floor((counts + float(BLK - 1)) * (1.0 / BLK)) * float(BLK)
    lane = jax.lax.broadcasted_iota(jnp.int32, (1, NUM_EXPERTS), 1)
    cum = padded
    sh = 1
    while sh < NUM_EXPERTS:
        cum = cum + jnp.where(lane >= sh, pltpu.roll(cum, sh, 1), 0.0)
        sh *= 2
    pad_off = cum - padded

    excl = jnp.concatenate(excls, axis=0)
    pos0 = jnp.sum(oh0 * (pad_off + excl), axis=1, keepdims=True)
    pos1 = jnp.sum(oh1 * (pad_off + excl + oh0), axis=1, keepdims=True)
    pos_ref[...] = jnp.concatenate([pos0, pos1], axis=1).astype(jnp.int32)
    wts_ref[...] = jnp.concatenate([w0, w1], axis=1)

    eye = (jax.lax.broadcasted_iota(jnp.int32, (NUM_EXPERTS, NUM_EXPERTS), 0) ==
           jax.lax.broadcasted_iota(jnp.int32, (NUM_EXPERTS, NUM_EXPERTS), 1))
    cum_col = jnp.sum(jnp.where(eye, cum, 0.0), axis=1, keepdims=True)
    bstart = (jax.lax.broadcasted_iota(jnp.int32, (1, 128), 1) * BLK).astype(jnp.float32)
    ge = (bstart >= cum_col).astype(jnp.int32)
    eids = jnp.minimum(jnp.sum(ge, axis=0, keepdims=True), NUM_EXPERTS - 1)
    eids_ref[...] = jnp.broadcast_to(eids, (8, 128))


def _router_meta(logits, nblk):
    t = logits.shape[0]
    return pl.pallas_call(
        functools.partial(_meta_kernel, nblk),
        out_shape=(
            jax.ShapeDtypeStruct((t, TOP_K), jnp.int32),
            jax.ShapeDtypeStruct((t, TOP_K), jnp.float32),
            jax.ShapeDtypeStruct((8, 128), jnp.int32),
        ),
    )(logits)


def _ffn_block_kernel(eid_ref, xs_ref, gp_ref, up_ref, dp_ref, out_ref):
    x = xs_ref[...].astype(jnp.bfloat16)
    gp = gp_ref[0].astype(jnp.bfloat16)
    up = up_ref[0].astype(jnp.bfloat16)
    dp = dp_ref[0].astype(jnp.bfloat16)
    g = lax.dot_general(x, gp, (((1,), (1,)), ((), ())),
                        preferred_element_type=jnp.float32)
    u = lax.dot_general(x, up, (((1,), (1,)), ((), ())),
                        preferred_element_type=jnp.float32)
    h = (g * jax.nn.sigmoid(g) * u).astype(jnp.bfloat16)
    out_ref[...] = lax.dot_general(h, dp, (((1,), (1,)), ((), ())),
                                   preferred_element_type=jnp.float32)


def _grouped_ffn(xs, gate_proj, up_proj, down_proj, expert_ids, nblk):
    grid_spec = pltpu.PrefetchScalarGridSpec(
        num_scalar_prefetch=1,
        grid=(nblk,),
        in_specs=[
            pl.BlockSpec((BLK, HIDDEN), lambda b, eid: (b, 0)),
            pl.BlockSpec((1, INTER, HIDDEN), lambda b, eid: (eid[b], 0, 0)),
            pl.BlockSpec((1, INTER, HIDDEN), lambda b, eid: (eid[b], 0, 0)),
            pl.BlockSpec((1, HIDDEN, INTER), lambda b, eid: (eid[b], 0, 0)),
        ],
        out_specs=pl.BlockSpec((BLK, HIDDEN), lambda b, eid: (b, 0)),
    )
    return pl.pallas_call(
        _ffn_block_kernel,
        grid_spec=grid_spec,
        out_shape=jax.ShapeDtypeStruct((xs.shape[0], HIDDEN), jnp.float32),
    )(expert_ids, xs, gate_proj, up_proj, down_proj)


def _sc_mesh():
    return plsc.VectorSubcoreMesh(core_axis_name="c", subcore_axis_name="s",
                                  num_cores=NC, num_subcores=NS)


def _sc_params():
    cp = pltpu.CompilerParams()
    if "needs_layout_passes" in pltpu.CompilerParams.__dataclass_fields__:
        cp = dataclasses.replace(cp, needs_layout_passes=False)
    return cp


def _scatter_body(tpw, x_hbm, idx_hbm, xs_hbm, rows_v, idx_v, sem):
    wid = lax.axis_index("c") * NS + lax.axis_index("s")
    base = wid * tpw
    pltpu.sync_copy(x_hbm.at[pl.ds(base, tpw)], rows_v)
    pltpu.sync_copy(idx_hbm.at[wid], idx_v)
    pltpu.async_copy(rows_v, xs_hbm.at[idx_v.at[0]], sem).wait()
    pltpu.async_copy(rows_v, xs_hbm.at[idx_v.at[1]], sem).wait()


def _dispatch_scatter(x, idx3, padded):
    t = x.shape[0]
    tpw = t // NW
    return pl.kernel(
        functools.partial(_scatter_body, tpw),
        out_type=jax.ShapeDtypeStruct((padded, HIDDEN), jnp.float32),
        mesh=_sc_mesh(),
        scratch_types=[
            pltpu.VMEM((tpw, HIDDEN), jnp.float32),
            pltpu.VMEM((TOP_K, tpw), jnp.int32),
            pltpu.SemaphoreType.DMA,
        ],
    )(x, idx3)


def _combine_body(tpw, chunk, y_hbm, idx_hbm, w_hbm, out_hbm,
                  buf0, buf1, idx_v, w_v, sem):
    wid = lax.axis_index("c") * NS + lax.axis_index("s")
    base = wid * tpw
    pltpu.sync_copy(idx_hbm.at[wid], idx_v)
    pltpu.sync_copy(w_hbm.at[wid], w_v)
    nchunks = tpw // chunk
    zero16 = jnp.zeros((16,), jnp.int32)
    one16 = jnp.ones((16,), jnp.int32)
    for c in range(nchunks):
        pltpu.async_copy(y_hbm.at[idx_v.at[0, pl.ds(c * chunk, chunk)]],
                         buf0, sem).wait()
        pltpu.async_copy(y_hbm.at[idx_v.at[1, pl.ds(c * chunk, chunk)]],
                         buf1, sem).wait()

        def row_body(r, _, c=c):
            rsplat = jnp.full((16,), c * chunk + r, jnp.int32)
            w0 = plsc.load_gather(w_v, [zero16, rsplat])
            w1 = plsc.load_gather(w_v, [one16, rsplat])
            for l in range(HIDDEN // 16):
                sl = pl.ds(l * 16, 16)
                buf0[r, sl] = buf0[r, sl] * w0 + buf1[r, sl] * w1
            return 0

        lax.fori_loop(0, chunk, row_body, 0)
        pltpu.sync_copy(buf0, out_hbm.at[pl.ds(base + c * chunk, chunk)])


def _combine(ybuf, idx3, w3, t):
    tpw = t // NW
    chunk = 32
    return pl.kernel(
        functools.partial(_combine_body, tpw, chunk),
        out_type=jax.ShapeDtypeStruct((t, HIDDEN), jnp.float32),
        mesh=_sc_mesh(),
        scratch_types=[
            pltpu.VMEM((chunk, HIDDEN), jnp.float32),
            pltpu.VMEM((chunk, HIDDEN), jnp.float32),
            pltpu.VMEM((TOP_K, tpw), jnp.int32),
            pltpu.VMEM((TOP_K, tpw), jnp.float32),
            pltpu.SemaphoreType.DMA,
        ],
        compiler_params=_sc_params(),
    )(ybuf, idx3, w3)


def kernel(hidden_states, gate_weight, gate_proj, up_proj, down_proj):
    b, s, h = hidden_states.shape
    x = hidden_states.reshape(-1, h)
    T = x.shape[0]
    nblk = T * TOP_K // BLK + NUM_EXPERTS
    padded = nblk * BLK

    logits = x @ gate_weight.T
    pos2, wts, eids_meta = _router_meta(logits, nblk)
    expert_ids = eids_meta[0, :nblk]

    idx3 = pos2.reshape(NW, T // NW, TOP_K).transpose(0, 2, 1)
    w3 = wts.reshape(NW, T // NW, TOP_K).transpose(0, 2, 1)

    xs = _dispatch_scatter(x, idx3, padded)
    ybuf = _grouped_ffn(xs, gate_proj, up_proj, down_proj, expert_ids, nblk)
    out = _combine(ybuf, idx3, w3, T)
    return out.reshape(b, s, h), logits


# speedup vs baseline: 2.6582x; 1.0843x over previous
import dataclasses
import functools

import jax
import jax.numpy as jnp
from jax import lax
from jax.experimental import pallas as pl
from jax.experimental.pallas import tpu as pltpu
from jax.experimental.pallas import tpu_sc as plsc

NUM_EXPERTS = 8
TOP_K = 2
HIDDEN = 1024
INTER = 768
BLK = 256

NC = 2
NS = 16
NW = NC * NS


def _meta_kernel(nblk, lg_ref, pos_ref, wts_ref, eids_ref):
    t = lg_ref.shape[0]
    lg = lg_ref[...]
    m = jnp.max(lg, axis=1, keepdims=True)
    ex = jnp.exp(lg - m)
    p = ex / jnp.sum(ex, axis=1, keepdims=True)

    idx = jax.lax.broadcasted_iota(jnp.int32, (t, NUM_EXPERTS), 1)
    m1 = jnp.max(p, axis=1, keepdims=True)
    i1 = jnp.min(jnp.where(p == m1, idx, NUM_EXPERTS), axis=1, keepdims=True)
    masked = jnp.where(idx == i1, -1.0, p)
    m2 = jnp.max(masked, axis=1, keepdims=True)
    i2 = jnp.min(jnp.where(masked == m2, idx, NUM_EXPERTS), axis=1, keepdims=True)
    wsum = m1 + m2
    w0 = m1 / wsum
    w1 = m2 / wsum

    oh0 = (idx == i1).astype(jnp.float32)
    oh1 = (idx == i2).astype(jnp.float32)
    ohsum = oh0 + oh1

    ci = jax.lax.broadcasted_iota(jnp.int32, (256, 256), 0)
    cj = jax.lax.broadcasted_iota(jnp.int32, (256, 256), 1)
    lstrict = (cj < ci).astype(jnp.bfloat16)
    nchunks = t // 256
    carry = jnp.zeros((1, NUM_EXPERTS), jnp.float32)
    excls = []
    for c in range(nchunks):
        oh_c = jax.lax.slice(ohsum, (c * 256, 0), ((c + 1) * 256, NUM_EXPERTS))
        excl_c = jax.lax.dot_general(lstrict, oh_c.astype(jnp.bfloat16),
                                     (((1,), (0,)), ((), ())),
                                     preferred_element_type=jnp.float32) + carry
        excls.append(excl_c)
        carry = carry + jnp.sum(oh_c, axis=0, keepdims=True)
    counts = carry

    padded = jnp.floor((counts + float(BLK - 1)) * (1.0 / BLK)) * float(BLK)
    lane = jax.lax.broadcasted_iota(jnp.int32, (1, NUM_EXPERTS), 1)
    cum = padded
    sh = 1
    while sh < NUM_EXPERTS:
        cum = cum + jnp.where(lane >= sh, pltpu.roll(cum, sh, 1), 0.0)
        sh *= 2
    pad_off = cum - padded

    excl = jnp.concatenate(excls, axis=0)
    pos0 = jnp.sum(oh0 * (pad_off + excl), axis=1, keepdims=True)
    pos1 = jnp.sum(oh1 * (pad_off + excl + oh0), axis=1, keepdims=True)
    pos_ref[...] = jnp.concatenate([pos0, pos1], axis=1).astype(jnp.int32)
    wts_ref[...] = jnp.concatenate([w0, w1], axis=1)

    eye = (jax.lax.broadcasted_iota(jnp.int32, (NUM_EXPERTS, NUM_EXPERTS), 0) ==
           jax.lax.broadcasted_iota(jnp.int32, (NUM_EXPERTS, NUM_EXPERTS), 1))
    cum_col = jnp.sum(jnp.where(eye, cum, 0.0), axis=1, keepdims=True)
    bstart = (jax.lax.broadcasted_iota(jnp.int32, (1, 128), 1) * BLK).astype(jnp.float32)
    ge = (bstart >= cum_col).astype(jnp.int32)
    eids = jnp.minimum(jnp.sum(ge, axis=0, keepdims=True), NUM_EXPERTS - 1)
    eids_ref[...] = jnp.broadcast_to(eids, (8, 128))


def _router_meta(logits, nblk):
    t = logits.shape[0]
    return pl.pallas_call(
        functools.partial(_meta_kernel, nblk),
        out_shape=(
            jax.ShapeDtypeStruct((t, TOP_K), jnp.int32),
            jax.ShapeDtypeStruct((t, TOP_K), jnp.float32),
            jax.ShapeDtypeStruct((8, 128), jnp.int32),
        ),
    )(logits)


def _ffn_block_kernel(mb_ref, xs_ref, gp_hbm, up_hbm, dp_hbm, out_ref,
                      wg, wu, wd, sem):
    b = pl.program_id(0)
    slot = mb_ref[2, b]

    @pl.when(b == 0)
    def _first():
        e0 = mb_ref[0, 0]
        pltpu.make_async_copy(gp_hbm.at[e0], wg.at[0], sem.at[0, 0]).start()
        pltpu.make_async_copy(up_hbm.at[e0], wu.at[0], sem.at[0, 1]).start()
        pltpu.make_async_copy(dp_hbm.at[e0], wd.at[0], sem.at[0, 2]).start()

    @pl.when(mb_ref[1, b] == 1)
    def _change():
        e = mb_ref[0, b]
        pltpu.make_async_copy(gp_hbm.at[e], wg.at[slot], sem.at[slot, 0]).wait()
        pltpu.make_async_copy(up_hbm.at[e], wu.at[slot], sem.at[slot, 1]).wait()
        pltpu.make_async_copy(dp_hbm.at[e], wd.at[slot], sem.at[slot, 2]).wait()

        @pl.when(mb_ref[4, b] == 1)
        def _issue_next():
            ne = mb_ref[3, b]
            ns = 1 - slot
            pltpu.make_async_copy(gp_hbm.at[ne], wg.at[ns], sem.at[ns, 0]).start()
            pltpu.make_async_copy(up_hbm.at[ne], wu.at[ns], sem.at[ns, 1]).start()
            pltpu.make_async_copy(dp_hbm.at[ne], wd.at[ns], sem.at[ns, 2]).start()

    x = xs_ref[...].astype(jnp.bfloat16)
    gp = wg[slot].astype(jnp.bfloat16)
    up = wu[slot].astype(jnp.bfloat16)
    dp = wd[slot].astype(jnp.bfloat16)
    g = lax.dot_general(x, gp, (((1,), (1,)), ((), ())),
                        preferred_element_type=jnp.float32)
    u = lax.dot_general(x, up, (((1,), (1,)), ((), ())),
                        preferred_element_type=jnp.float32)
    h = (g * jax.nn.sigmoid(g) * u).astype(jnp.bfloat16)
    out_ref[...] = lax.dot_general(h, dp, (((1,), (1,)), ((), ())),
                                   preferred_element_type=jnp.float32)


def _grouped_ffn(xs, gate_proj, up_proj, down_proj, meta_blk, nblk):
    hbm = pl.BlockSpec(memory_space=pltpu.MemorySpace.HBM)
    grid_spec = pltpu.PrefetchScalarGridSpec(
        num_scalar_prefetch=1,
        grid=(nblk,),
        in_specs=[
            pl.BlockSpec((BLK, HIDDEN), lambda b, mb: (b, 0)),
            hbm, hbm, hbm,
        ],
        out_specs=pl.BlockSpec((BLK, HIDDEN), lambda b, mb: (b, 0)),
        scratch_shapes=[
            pltpu.VMEM((2, INTER, HIDDEN), jnp.float32),
            pltpu.VMEM((2, INTER, HIDDEN), jnp.float32),
            pltpu.VMEM((2, HIDDEN, INTER), jnp.float32),
            pltpu.SemaphoreType.DMA((2, 3)),
        ],
    )
    return pl.pallas_call(
        _ffn_block_kernel,
        grid_spec=grid_spec,
        out_shape=jax.ShapeDtypeStruct((xs.shape[0], HIDDEN), jnp.float32),
    )(meta_blk, xs, gate_proj, up_proj, down_proj)


def _sc_mesh():
    return plsc.VectorSubcoreMesh(core_axis_name="c", subcore_axis_name="s",
                                  num_cores=NC, num_subcores=NS)


def _sc_params():
    cp = pltpu.CompilerParams()
    if "needs_layout_passes" in pltpu.CompilerParams.__dataclass_fields__:
        cp = dataclasses.replace(cp, needs_layout_passes=False)
    return cp


def _scatter_body(tpw, x_hbm, idx_hbm, xs_hbm, rows_v, idx_v, sem):
    wid = lax.axis_index("c") * NS + lax.axis_index("s")
    base = wid * tpw
    pltpu.sync_copy(x_hbm.at[pl.ds(base, tpw)], rows_v)
    pltpu.sync_copy(idx_hbm.at[wid], idx_v)
    pltpu.async_copy(rows_v, xs_hbm.at[idx_v.at[0]], sem).wait()
    pltpu.async_copy(rows_v, xs_hbm.at[idx_v.at[1]], sem).wait()


def _dispatch_scatter(x, idx3, padded):
    t = x.shape[0]
    tpw = t // NW
    return pl.kernel(
        functools.partial(_scatter_body, tpw),
        out_type=jax.ShapeDtypeStruct((padded, HIDDEN), jnp.float32),
        mesh=_sc_mesh(),
        scratch_types=[
            pltpu.VMEM((tpw, HIDDEN), jnp.float32),
            pltpu.VMEM((TOP_K, tpw), jnp.int32),
            pltpu.SemaphoreType.DMA,
        ],
    )(x, idx3)


def _combine_body(tpw, chunk, y_hbm, idx_hbm, w_hbm, out_hbm,
                  buf0, buf1, idx_v, w_v, sem):
    wid = lax.axis_index("c") * NS + lax.axis_index("s")
    base = wid * tpw
    pltpu.sync_copy(idx_hbm.at[wid], idx_v)
    pltpu.sync_copy(w_hbm.at[wid], w_v)
    nchunks = tpw // chunk
    zero16 = jnp.zeros((16,), jnp.int32)
    one16 = jnp.ones((16,), jnp.int32)
    for c in range(nchunks):
        pltpu.async_copy(y_hbm.at[idx_v.at[0, pl.ds(c * chunk, chunk)]],
                         buf0, sem).wait()
        pltpu.async_copy(y_hbm.at[idx_v.at[1, pl.ds(c * chunk, chunk)]],
                         buf1, sem).wait()

        def row_body(r, _, c=c):
            rsplat = jnp.full((16,), c * chunk + r, jnp.int32)
            w0 = plsc.load_gather(w_v, [zero16, rsplat])
            w1 = plsc.load_gather(w_v, [one16, rsplat])
            for l in range(HIDDEN // 16):
                sl = pl.ds(l * 16, 16)
                buf0[r, sl] = buf0[r, sl] * w0 + buf1[r, sl] * w1
            return 0

        lax.fori_loop(0, chunk, row_body, 0)
        pltpu.sync_copy(buf0, out_hbm.at[pl.ds(base + c * chunk, chunk)])


def _combine(ybuf, idx3, w3, t):
    tpw = t // NW
    chunk = 32
    return pl.kernel(
        functools.partial(_combine_body, tpw, chunk),
        out_type=jax.ShapeDtypeStruct((t, HIDDEN), jnp.float32),
        mesh=_sc_mesh(),
        scratch_types=[
            pltpu.VMEM((chunk, HIDDEN), jnp.float32),
            pltpu.VMEM((chunk, HIDDEN), jnp.float32),
            pltpu.VMEM((TOP_K, tpw), jnp.int32),
            pltpu.VMEM((TOP_K, tpw), jnp.float32),
            pltpu.SemaphoreType.DMA,
        ],
        compiler_params=_sc_params(),
    )(ybuf, idx3, w3)


def kernel(hidden_states, gate_weight, gate_proj, up_proj, down_proj):
    b, s, h = hidden_states.shape
    x = hidden_states.reshape(-1, h)
    T = x.shape[0]
    nblk = T * TOP_K // BLK + NUM_EXPERTS
    padded = nblk * BLK

    logits = x @ gate_weight.T
    pos2, wts, eids_meta = _router_meta(logits, nblk)
    eids = eids_meta[0, :nblk]

    chg = jnp.concatenate([jnp.ones((1,), jnp.int32),
                           (eids[1:] != eids[:-1]).astype(jnp.int32)])
    slot = (jnp.cumsum(chg) - 1) % 2
    idxs = jnp.arange(nblk, dtype=jnp.int32)
    cand = jnp.where(chg == 1, idxs, nblk + 1)
    sufmin = lax.associative_scan(jnp.minimum, cand, reverse=True)
    nxt_idx = jnp.concatenate([sufmin[1:], jnp.full((1,), nblk + 1, jnp.int32)])
    hasnxt = (nxt_idx < nblk).astype(jnp.int32)
    nxt_e = eids[jnp.minimum(nxt_idx, nblk - 1)]
    meta_blk = jnp.stack([eids, chg, slot, nxt_e, hasnxt])

    idx3 = pos2.reshape(NW, T // NW, TOP_K).transpose(0, 2, 1)
    w3 = wts.reshape(NW, T // NW, TOP_K).transpose(0, 2, 1)

    xs = _dispatch_scatter(x, idx3, padded)
    ybuf = _grouped_ffn(xs, gate_proj, up_proj, down_proj, meta_blk, nblk)
    out = _combine(ybuf, idx3, w3, T)
    return out.reshape(b, s, h), logits
